# baseline (speedup 1.0000x reference)
import functools

import jax
import jax.numpy as jnp
from jax import lax
from jax.experimental import pallas as pl
from jax.experimental.pallas import tpu as pltpu
from jax.experimental.pallas import tpu_sc as plsc

S = 64.0
MEAN = 0.35
SIGMA = 0.0125

N_ROWS = 1024
N_COLS = 100000
NC, NS, L = 2, 16, 16
NW = NC * NS
RPW = N_ROWS // NW
RG = 16
NRG = RPW // RG
CHUNK = 6400
NCH = 15
TAIL = 3968
SC_COLS = NCH * CHUNK + TAIL
TC_COLS = N_COLS - SC_COLS

_mesh = plsc.VectorSubcoreMesh(
    core_axis_name="c", subcore_axis_name="s", num_cores=NC, num_subcores=NS
)


@functools.partial(
    pl.kernel,
    out_type=jax.ShapeDtypeStruct((N_ROWS, N_COLS), jnp.float32),
    mesh=_mesh,
    scratch_types=[
        pltpu.VMEM((RPW,), jnp.int32),
        pltpu.VMEM((RPW,), jnp.float32),
        pltpu.VMEM((RG, CHUNK), jnp.float32),
    ],
    compiler_params=pltpu.CompilerParams(needs_layout_passes=False),
)
def _sc_scale(logits_hbm, labels_hbm, ela_hbm, out_hbm, lab_v, ela_v, buf):
    wid = lax.axis_index("s") * NC + lax.axis_index("c")
    base_row = wid * RPW
    pltpu.sync_copy(labels_hbm.at[pl.ds(base_row, RPW)], lab_v)
    pltpu.sync_copy(ela_hbm.at[pl.ds(base_row, RPW)], ela_v)
    rows16 = lax.iota(jnp.int32, L)

    def process_block(g, lab16, ela16, c0, width, n_vec):
        row0 = pl.multiple_of(base_row + g * RG, RG)
        src = logits_hbm.at[pl.ds(row0, RG), pl.ds(c0, width)]
        dst = buf.at[:, pl.ds(0, width)] if width != CHUNK else buf
        pltpu.sync_copy(src, dst)

        def vec_loop(j, carry):
            o = j * L
            for r in range(RG):
                buf[r, pl.ds(o, L)] = buf[r, pl.ds(o, L)] * S
            return carry

        lax.fori_loop(0, n_vec, vec_loop, 0, unroll=2)

        off = lab16 - c0
        m = (off >= 0) & (off < width)
        offc = jnp.clip(off, 0, width - 1)
        v = plsc.load_gather(buf, [rows16, offc], mask=m)
        plsc.store_scatter(buf, [rows16, offc], v - ela16, mask=m)

        pltpu.sync_copy(dst, out_hbm.at[pl.ds(row0, RG), pl.ds(c0, width)])

    def rg_loop(g, carry):
        lab16 = lab_v[pl.ds(pl.multiple_of(g * RG, RG), L)]
        ela16 = ela_v[pl.ds(pl.multiple_of(g * RG, RG), L)]

        def chunk_loop(b, carry2):
            c0 = pl.multiple_of(b * CHUNK, 128)
            process_block(g, lab16, ela16, c0, CHUNK, CHUNK // L)
            return carry2

        lax.fori_loop(0, NCH, chunk_loop, 0)
        process_block(g, lab16, ela16, NCH * CHUNK, TAIL, TAIL // L)
        return carry

    lax.fori_loop(0, NRG, rg_loop, 0)


def _tc_tail(sc_ref, logits_ref, labels_ref, ela_ref, out_ref):
    cols = SC_COLS + jax.lax.broadcasted_iota(jnp.int32, (N_ROWS, 128), 1)
    hit = cols == labels_ref[:, :]
    out_ref[:, :] = logits_ref[:, :] * S - jnp.where(hit, ela_ref[:, :], 0.0)


def kernel(logits, labels):
    ekey = jax.random.key(42)
    ela_s = (MEAN + SIGMA * jax.random.normal(ekey, (N_ROWS,), dtype=jnp.float32)) * S
    sc_out = _sc_scale(logits, labels, ela_s)

    ela2 = ela_s.reshape(N_ROWS, 1)
    labels2 = labels.reshape(N_ROWS, 1)
    tail_block = pl.BlockSpec((N_ROWS, 128), lambda i: (0, SC_COLS // 128))
    return pl.pallas_call(
        _tc_tail,
        grid=(1,),
        in_specs=[
            tail_block,
            tail_block,
            pl.BlockSpec((N_ROWS, 1), lambda i: (0, 0)),
            pl.BlockSpec((N_ROWS, 1), lambda i: (0, 0)),
        ],
        out_specs=tail_block,
        out_shape=jax.ShapeDtypeStruct((N_ROWS, N_COLS), jnp.float32),
        input_output_aliases={0: 0},
    )(sc_out, logits, labels2, ela2)


# speedup vs baseline: 1.0751x; 1.0751x over previous
import functools

import jax
import jax.numpy as jnp
from jax import lax
from jax.experimental import pallas as pl
from jax.experimental.pallas import tpu as pltpu
from jax.experimental.pallas import tpu_sc as plsc

S = 64.0
MEAN = 0.35
SIGMA = 0.0125

N_ROWS = 1024
N_COLS = 100000
NC, NS, L = 2, 16, 16
NW = NC * NS
RPW = N_ROWS // NW
RG = 16
NRG = RPW // RG
CHUNK = 2048
NCH = 48
SC_COLS = NCH * CHUNK
NIT = NRG * NCH
NSLOT = 3
TC_COLS = 2048
NVEC = CHUNK // L

_mesh = plsc.VectorSubcoreMesh(
    core_axis_name="c", subcore_axis_name="s", num_cores=NC, num_subcores=NS
)


@functools.partial(
    pl.kernel,
    out_type=jax.ShapeDtypeStruct((N_ROWS, N_COLS), jnp.float32),
    mesh=_mesh,
    scratch_types=[
        pltpu.VMEM((RPW,), jnp.int32),
        pltpu.VMEM((RPW,), jnp.float32),
        pltpu.VMEM((NSLOT, RG, CHUNK), jnp.float32),
        pltpu.SemaphoreType.DMA,
        pltpu.SemaphoreType.DMA,
    ],
    compiler_params=pltpu.CompilerParams(needs_layout_passes=False),
)
def _sc_scale(logits_hbm, labels_hbm, ela_hbm, out_hbm, lab_v, ela_v, buf,
              in_sem, out_sem):
    wid = lax.axis_index("s") * NC + lax.axis_index("c")
    base_row = wid * RPW
    pltpu.sync_copy(labels_hbm.at[pl.ds(base_row, RPW)], lab_v)
    pltpu.sync_copy(ela_hbm.at[pl.ds(base_row, RPW)], ela_v)
    rows16 = lax.iota(jnp.int32, L)

    def chunk_src(t):
        g = t // NCH
        b = t - g * NCH
        row0 = pl.multiple_of(base_row + g * RG, RG)
        c0 = pl.multiple_of(b * CHUNK, 128)
        return logits_hbm.at[pl.ds(row0, RG), pl.ds(c0, CHUNK)]

    def chunk_dst(t):
        g = t // NCH
        b = t - g * NCH
        row0 = pl.multiple_of(base_row + g * RG, RG)
        c0 = pl.multiple_of(b * CHUNK, 128)
        return out_hbm.at[pl.ds(row0, RG), pl.ds(c0, CHUNK)]

    pltpu.async_copy(chunk_src(0), buf.at[0], in_sem)

    def step(t, k):
        kn = (k + 1) % NSLOT
        @pl.when(t >= 2)
        def _():
            tp = jnp.maximum(t - 2, 0)
            pltpu.make_async_copy(buf.at[kn], chunk_dst(tp), out_sem).wait()

        @pl.when(t + 1 < NIT)
        def _():
            tn = jnp.minimum(t + 1, NIT - 1)
            pltpu.async_copy(chunk_src(tn), buf.at[kn], in_sem)

        pltpu.make_async_copy(chunk_src(t), buf.at[k], in_sem).wait()

        def vec_loop(j, carry):
            o = j * L
            for r in range(RG):
                buf[k, r, pl.ds(o, L)] = buf[k, r, pl.ds(o, L)] * S
            return carry

        lax.fori_loop(0, NVEC, vec_loop, 0, unroll=2)

        g = t // NCH
        c0 = (t - g * NCH) * CHUNK
        goff = pl.multiple_of(g * RG, RG)
        lab16 = lab_v[pl.ds(goff, L)]
        ela16 = ela_v[pl.ds(goff, L)]
        off = lab16 - c0
        m = (off >= 0) & (off < CHUNK)
        offc = jnp.clip(off, 0, CHUNK - 1)
        slot = buf.at[k]
        v = plsc.load_gather(slot, [rows16, offc], mask=m)
        plsc.store_scatter(slot, [rows16, offc], v - ela16, mask=m)

        pltpu.async_copy(buf.at[k], chunk_dst(t), out_sem)

    def ring_loop(tt, carry):
        for kk in range(NSLOT):
            step(tt * NSLOT + kk, kk)
        return carry

    lax.fori_loop(0, NIT // NSLOT, ring_loop, 0)

    pltpu.make_async_copy(buf.at[(NIT - 2) % NSLOT], chunk_dst(NIT - 2), out_sem).wait()
    pltpu.make_async_copy(buf.at[(NIT - 1) % NSLOT], chunk_dst(NIT - 1), out_sem).wait()


def _tc_tail(sc_ref, logits_ref, labels_ref, ela_ref, out_ref):
    cols = SC_COLS + jax.lax.broadcasted_iota(jnp.int32, (N_ROWS, TC_COLS), 1)
    hit = cols == labels_ref[:, :]
    out_ref[:, :] = logits_ref[:, :] * S - jnp.where(hit, ela_ref[:, :], 0.0)


def kernel(logits, labels):
    ekey = jax.random.key(42)
    ela_s = (MEAN + SIGMA * jax.random.normal(ekey, (N_ROWS,), dtype=jnp.float32)) * S
    sc_out = _sc_scale(logits, labels, ela_s)

    ela2 = ela_s.reshape(N_ROWS, 1)
    labels2 = labels.reshape(N_ROWS, 1)
    tail_block = pl.BlockSpec((N_ROWS, TC_COLS), lambda i: (0, SC_COLS // TC_COLS))
    return pl.pallas_call(
        _tc_tail,
        grid=(1,),
        in_specs=[
            tail_block,
            tail_block,
            pl.BlockSpec((N_ROWS, 1), lambda i: (0, 0)),
            pl.BlockSpec((N_ROWS, 1), lambda i: (0, 0)),
        ],
        out_specs=tail_block,
        out_shape=jax.ShapeDtypeStruct((N_ROWS, N_COLS), jnp.float32),
        input_output_aliases={0: 0},
    )(sc_out, logits, labels2, ela2)


# speedup vs baseline: 6.5633x; 6.1046x over previous
import functools

import jax
import jax.numpy as jnp
from jax.experimental import pallas as pl

S = 64.0
MEAN = 0.35
SIGMA = 0.0125

N_ROWS = 1024
N_COLS = 100000
BR = 2048


def _body(lab_ref, ela_ref, lt_ref, out_ref):
    r0 = pl.program_id(0) * BR
    rows = r0 + jax.lax.broadcasted_iota(jnp.int32, (BR, N_ROWS), 0)
    hit = rows == lab_ref[:, :]
    out_ref[:, :] = lt_ref[:, :] * S - jnp.where(hit, ela_ref[:, :], 0.0)


def kernel(logits, labels):
    ekey = jax.random.key(42)
    ela_s = (MEAN + SIGMA * jax.random.normal(ekey, (N_ROWS,), dtype=jnp.float32)) * S
    lt = jnp.swapaxes(logits, 0, 1)
    lab2 = labels.reshape(1, N_ROWS)
    ela2 = ela_s.reshape(1, N_ROWS)

    grid = (pl.cdiv(N_COLS, BR),)
    out_t = pl.pallas_call(
        _body,
        grid=grid,
        in_specs=[
            pl.BlockSpec((1, N_ROWS), lambda i: (0, 0)),
            pl.BlockSpec((1, N_ROWS), lambda i: (0, 0)),
            pl.BlockSpec((BR, N_ROWS), lambda i: (i, 0)),
        ],
        out_specs=pl.BlockSpec((BR, N_ROWS), lambda i: (i, 0)),
        out_shape=jax.ShapeDtypeStruct((N_COLS, N_ROWS), jnp.float32),
    )(lab2, ela2, lt)
    return jnp.swapaxes(out_t, 0, 1)
